# baseline (speedup 1.0000x reference)
import jax
import jax.numpy as jnp
import numpy as np
from jax import lax
from jax.experimental import pallas as pl
from jax.experimental.pallas import tpu as pltpu
from jax.experimental.pallas import tpu_sc as plsc

NC = 2
NS = 16
NW = NC * NS
LANES = 16

NPAD = 10240
STRIPE = NPAD // NS
ECH = 64
ENCH = 160
EPW = ENCH * ECH
EP = NW * EPW
NBUF = 4
DCH = 128
DNCH = 26
DPW = DNCH * DCH
ELP = NW * DPW

_SC_PARAMS = dict(
    compiler_params=pltpu.CompilerParams(
        use_tc_tiling_on_sc=False, needs_layout_passes=False),
)


def _mesh():
    return plsc.VectorSubcoreMesh(core_axis_name="c", subcore_axis_name="s")


def _expand_perm(d):
    p = np.concatenate([np.arange(0, 32, 2), np.arange(1, 32, 2)])
    p_full = np.concatenate([b * 32 + p for b in range(d // 32)])
    return np.argsort(p_full)


def _pack_bf16(y):
    n, d = y.shape
    yb = y.astype(jnp.bfloat16).reshape(n, d // 2, 2)
    return jax.lax.bitcast_convert_type(yb, jnp.int32)


def _tc1_body(x_ref, wl_ref, wr_ref, b_ref, y_ref, r_ref):
    xv = x_ref[...]
    y_ref[...] = jnp.dot(xv, wl_ref[...], preferred_element_type=jnp.float32)
    r_ref[...] = (
        jnp.dot(xv, wr_ref[...], preferred_element_type=jnp.float32) + b_ref[...]
    )


def _tc1(x, W_l, W_r, b, br=2000):
    n, d = x.shape
    dh = W_l.shape[1]
    grid = n // br
    return pl.pallas_call(
        _tc1_body,
        grid=(grid,),
        in_specs=[
            pl.BlockSpec((br, d), lambda i: (i, 0)),
            pl.BlockSpec((d, dh), lambda i: (0, 0)),
            pl.BlockSpec((d, dh), lambda i: (0, 0)),
            pl.BlockSpec((1, dh), lambda i: (0, 0)),
        ],
        out_specs=[
            pl.BlockSpec((br, dh), lambda i: (i, 0)),
            pl.BlockSpec((br, dh), lambda i: (i, 0)),
        ],
        out_shape=[
            jax.ShapeDtypeStruct((n, dh), jnp.float32),
            jax.ShapeDtypeStruct((n, dh), jnp.float32),
        ],
    )(x, W_l, W_r, b.reshape(1, dh))


def _tc2_body(aggA, aggB, dgA, dgB, r1_ref, wl_ref, wr_ref, b_ref, y2_ref, r2_ref):
    deg = dgA[...] + dgB[...]
    inv = 1.0 / jnp.maximum(deg, 1.0)
    h = jnp.maximum((aggA[...] + aggB[...]) * inv + r1_ref[...], 0.0)
    y2_ref[...] = jnp.dot(h, wl_ref[...], preferred_element_type=jnp.float32)
    r2_ref[...] = (
        jnp.dot(h, wr_ref[...], preferred_element_type=jnp.float32) + b_ref[...]
    )


def _tc2(agg, deg2d, r1, W_l2, W_r2, b_l2, br=2048):
    dh = r1.shape[1]
    do = W_l2.shape[1]
    grid = NPAD // br
    off = NPAD // br
    return pl.pallas_call(
        _tc2_body,
        grid=(grid,),
        in_specs=[
            pl.BlockSpec((br, dh), lambda i: (i, 0)),
            pl.BlockSpec((br, dh), lambda i: (i + off, 0)),
            pl.BlockSpec((br, 1), lambda i: (i, 0)),
            pl.BlockSpec((br, 1), lambda i: (i + off, 0)),
            pl.BlockSpec((br, dh), lambda i: (i, 0)),
            pl.BlockSpec((dh, do), lambda i: (0, 0)),
            pl.BlockSpec((dh, do), lambda i: (0, 0)),
            pl.BlockSpec((1, do), lambda i: (0, 0)),
        ],
        out_specs=[
            pl.BlockSpec((br, do), lambda i: (i, 0)),
            pl.BlockSpec((br, do), lambda i: (i, 0)),
        ],
        out_shape=[
            jax.ShapeDtypeStruct((NPAD, do), jnp.float32),
            jax.ShapeDtypeStruct((NPAD, do), jnp.float32),
        ],
    )(agg, agg, deg2d, deg2d, r1, W_l2, W_r2, b_l2.reshape(1, do))


def _tc3_body(aggA, aggB, dgA, dgB, r2_ref, z_ref):
    deg = dgA[...] + dgB[...]
    inv = 1.0 / jnp.maximum(deg, 1.0)
    z_ref[...] = (aggA[...] + aggB[...]) * inv + r2_ref[...]


def _tc3(agg, deg2d, r2, br=2048):
    do = r2.shape[1]
    grid = NPAD // br
    off = NPAD // br
    return pl.pallas_call(
        _tc3_body,
        grid=(grid,),
        in_specs=[
            pl.BlockSpec((br, do), lambda i: (i, 0)),
            pl.BlockSpec((br, do), lambda i: (i + off, 0)),
            pl.BlockSpec((br, 1), lambda i: (i, 0)),
            pl.BlockSpec((br, 1), lambda i: (i + off, 0)),
            pl.BlockSpec((br, do), lambda i: (i, 0)),
        ],
        out_specs=pl.BlockSpec((br, do), lambda i: (i, 0)),
        out_shape=jax.ShapeDtypeStruct((NPAD, do), jnp.float32),
    )(agg, agg, deg2d, deg2d, r2)


def _sc_degree(packed):
    def body(packed_hbm, deg_out, hist_sh, packed_v, hist_v, part_v, deg_v):
        c = lax.axis_index("c")
        s = lax.axis_index("s")
        wid = c * NS + s
        row0 = s * STRIPE
        pltpu.sync_copy(packed_hbm.at[wid], packed_v)

        zeros16 = jnp.zeros((LANES,), jnp.float32)

        def zero_hist(g, carry):
            hist_v[pl.ds(g * LANES, LANES)] = zeros16
            return carry

        lax.fori_loop(0, NPAD // LANES, zero_hist, 0)

        ones16 = jnp.full((LANES,), 1.0, jnp.float32)

        def count(g, carry):
            p = packed_v[pl.ds(g * LANES, LANES)]
            d_idx = lax.shift_right_logical(p, 16)
            plsc.addupdate_scatter(hist_v, [d_idx], ones16)
            return carry

        lax.fori_loop(0, EPW // LANES, count, 0)

        pltpu.sync_copy(hist_v, hist_sh.at[s])
        plsc.subcore_barrier()
        for t in range(NS):
            pltpu.sync_copy(hist_sh.at[t, pl.ds(row0, STRIPE)], part_v.at[t])

        def merge(g, carry):
            acc = part_v[0, pl.ds(g * LANES, LANES)]
            for t in range(1, NS):
                acc = acc + part_v[t, pl.ds(g * LANES, LANES)]
            deg_v[pl.ds(g * LANES, LANES)] = acc
            return carry

        lax.fori_loop(0, STRIPE // LANES, merge, 0)
        pltpu.sync_copy(deg_v, deg_out.at[pl.ds(c * NPAD + row0, STRIPE)])

    fn = pl.kernel(
        body,
        out_type=jax.ShapeDtypeStruct((NC * NPAD,), jnp.float32),
        mesh=_mesh(),
        scratch_types=[
            pltpu.VMEM_SHARED((NS, NPAD), jnp.float32),
            pltpu.VMEM((EPW,), jnp.int32),
            pltpu.VMEM((NPAD,), jnp.float32),
            pltpu.VMEM((NS, STRIPE), jnp.float32),
            pltpu.VMEM((STRIPE,), jnp.float32),
        ],
        **_SC_PARAMS,
    )
    return fn(packed)


def _sc_aggregate(ypk, packed, zeros_d):
    n, dw = ypk.shape
    d = 2 * dw

    def body(y_hbm, packed_hbm, z_hbm, agg_out, agg_sh, packed_v,
             sidx, didx, rb0, rb1, rb2, rb3, f20, f21,
             gs0, gs1, gs2, gs3, ss0, ss1, ss2, ss3):
        rb = [rb0, rb1, rb2, rb3]
        f2 = [f20, f21]
        gs = [gs0, gs1, gs2, gs3]
        ss = [ss0, ss1, ss2, ss3]
        c = lax.axis_index("c")
        s = lax.axis_index("s")
        wid = c * NS + s
        row0 = s * STRIPE
        pltpu.sync_copy(packed_hbm.at[wid], packed_v)

        def unpack(jc, t):
            for g in range(ECH // LANES):
                p = packed_v[pl.ds(jc * ECH + g * LANES, LANES)]
                sidx[t, pl.ds(g * LANES, LANES)] = p & 0xFFFF
                didx[t, pl.ds(g * LANES, LANES)] = lax.shift_right_logical(p, 16)

        def gather(jc_unused, t):
            pltpu.async_copy(y_hbm.at[sidx.at[t]], rb[t], gs[t])

        def wait_gather(t):
            pltpu.make_async_copy(y_hbm.at[sidx.at[t]], rb[t], gs[t]).wait()

        def convert(t):
            fb = f2[t % 2]
            hi = jnp.int32(-65536)

            def crow(r, carry):
                for q in range(dw // LANES):
                    w = rb[t][r, pl.ds(q * LANES, LANES)]
                    fb[r, pl.ds(q * 2 * LANES, LANES)] = plsc.bitcast(
                        jnp.left_shift(w, 16), jnp.float32)
                    fb[r, pl.ds(q * 2 * LANES + LANES, LANES)] = plsc.bitcast(
                        w & hi, jnp.float32)
                return carry

            lax.fori_loop(0, ECH, crow, 0)

        def scatter(t):
            pltpu.async_copy(f2[t % 2], agg_sh.at[didx.at[t]], ss[t],
                             add=True)

        def wait_scatter(t):
            pltpu.make_async_copy(f2[t % 2], agg_sh.at[didx.at[t]],
                                  ss[t]).wait()

        unpack(0, 0)
        unpack(1, 1)
        gather(0, 0)
        gather(1, 1)
        pltpu.sync_copy(z_hbm.at[pl.ds(row0, STRIPE)],
                        agg_sh.at[pl.ds(row0, STRIPE)])
        plsc.subcore_barrier()

        for b in range(NBUF):
            b2 = (b + 2) % NBUF
            wait_gather(b)
            if b >= 2:
                wait_scatter(b2)
            convert(b)
            scatter(b)
            unpack(b + 2, b2)
            gather(b + 2, b2)

        def visit(k, carry):
            for b in range(NBUF):
                j = NBUF * k + b
                b2 = (b + 2) % NBUF
                wait_gather(b)
                wait_scatter(b2)
                convert(b)
                scatter(b)
                jn = jnp.minimum(j + 2, ENCH - 1)
                unpack(jn, b2)
                gather(jn, b2)
            return carry

        lax.fori_loop(1, ENCH // NBUF, visit, 0)

        wait_gather(0)
        wait_gather(1)
        wait_scatter(2)
        wait_scatter(3)
        plsc.subcore_barrier()
        pltpu.sync_copy(agg_sh.at[pl.ds(row0, STRIPE)],
                        agg_out.at[pl.ds(c * NPAD + row0, STRIPE)])

    fn = pl.kernel(
        body,
        out_type=jax.ShapeDtypeStruct((NC * NPAD, d), jnp.float32),
        mesh=_mesh(),
        scratch_types=[
            pltpu.VMEM_SHARED((NPAD, d), jnp.float32),
            pltpu.VMEM((EPW,), jnp.int32),
            pltpu.VMEM((NBUF, ECH), jnp.int32),
            pltpu.VMEM((NBUF, ECH), jnp.int32),
            pltpu.VMEM((ECH, dw), jnp.int32),
            pltpu.VMEM((ECH, dw), jnp.int32),
            pltpu.VMEM((ECH, dw), jnp.int32),
            pltpu.VMEM((ECH, dw), jnp.int32),
            pltpu.VMEM((ECH, d), jnp.float32),
            pltpu.VMEM((ECH, d), jnp.float32),
            pltpu.SemaphoreType.DMA,
            pltpu.SemaphoreType.DMA,
            pltpu.SemaphoreType.DMA,
            pltpu.SemaphoreType.DMA,
            pltpu.SemaphoreType.DMA,
            pltpu.SemaphoreType.DMA,
            pltpu.SemaphoreType.DMA,
            pltpu.SemaphoreType.DMA,
        ],
        **_SC_PARAMS,
    )
    return fn(ypk, packed, zeros_d)


def _sc_decode(zpk, ia, ib):
    n, dw = zpk.shape

    def body(z_hbm, ia_hbm, ib_hbm, out_hbm, ia_v, ib_v,
             za0, zb0, za1, zb1, out_v, gs0, gs1):
        c = lax.axis_index("c")
        s = lax.axis_index("s")
        wid = c * NS + s
        pltpu.sync_copy(ia_hbm.at[pl.ds(wid * DPW, DPW)], ia_v)
        pltpu.sync_copy(ib_hbm.at[pl.ds(wid * DPW, DPW)], ib_v)
        pltpu.async_copy(z_hbm.at[ia_v.at[pl.ds(0, DCH)]], za0, gs0)
        pltpu.async_copy(z_hbm.at[ib_v.at[pl.ds(0, DCH)]], zb0, gs0)

        def compute(j, za_v, zb_v):
            base = j * DCH
            hi = jnp.int32(-65536)
            for g in range(DCH // LANES):
                rws = g * LANES + lax.iota(jnp.int32, LANES)
                acc = jnp.zeros((LANES,), jnp.float32)
                for q in range(dw):
                    cols = jnp.full((LANES,), q, jnp.int32)
                    wa = plsc.load_gather(za_v, [rws, cols])
                    wb = plsc.load_gather(zb_v, [rws, cols])
                    ae = plsc.bitcast(jnp.left_shift(wa, 16), jnp.float32)
                    be = plsc.bitcast(jnp.left_shift(wb, 16), jnp.float32)
                    ao = plsc.bitcast(wa & hi, jnp.float32)
                    bo = plsc.bitcast(wb & hi, jnp.float32)
                    acc = acc + ae * be + ao * bo
                out_v[pl.ds(base + g * LANES, LANES)] = acc

        def pair(k, carry):
            j0 = 2 * k
            pltpu.make_async_copy(z_hbm.at[ia_v.at[pl.ds(0, DCH)]], za0,
                                  gs0).wait()
            pltpu.make_async_copy(z_hbm.at[ib_v.at[pl.ds(0, DCH)]], zb0,
                                  gs0).wait()
            o1 = (j0 + 1) * DCH
            pltpu.async_copy(z_hbm.at[ia_v.at[pl.ds(o1, DCH)]], za1, gs1)
            pltpu.async_copy(z_hbm.at[ib_v.at[pl.ds(o1, DCH)]], zb1, gs1)
            compute(j0, za0, zb0)
            pltpu.make_async_copy(z_hbm.at[ia_v.at[pl.ds(o1, DCH)]], za1,
                                  gs1).wait()
            pltpu.make_async_copy(z_hbm.at[ib_v.at[pl.ds(o1, DCH)]], zb1,
                                  gs1).wait()
            on = jnp.minimum(j0 + 2, DNCH - 1) * DCH
            pltpu.async_copy(z_hbm.at[ia_v.at[pl.ds(on, DCH)]], za0, gs0)
            pltpu.async_copy(z_hbm.at[ib_v.at[pl.ds(on, DCH)]], zb0, gs0)
            compute(j0 + 1, za1, zb1)
            return carry

        lax.fori_loop(0, DNCH // 2, pair, 0)
        pltpu.make_async_copy(z_hbm.at[ia_v.at[pl.ds(0, DCH)]], za0,
                              gs0).wait()
        pltpu.make_async_copy(z_hbm.at[ib_v.at[pl.ds(0, DCH)]], zb0,
                              gs0).wait()
        pltpu.sync_copy(out_v, out_hbm.at[pl.ds(wid * DPW, DPW)])

    fn = pl.kernel(
        body,
        out_type=jax.ShapeDtypeStruct((ELP,), jnp.float32),
        mesh=_mesh(),
        scratch_types=[
            pltpu.VMEM((DPW,), jnp.int32),
            pltpu.VMEM((DPW,), jnp.int32),
            pltpu.VMEM((DCH, dw), jnp.int32),
            pltpu.VMEM((DCH, dw), jnp.int32),
            pltpu.VMEM((DCH, dw), jnp.int32),
            pltpu.VMEM((DCH, dw), jnp.int32),
            pltpu.VMEM((DPW,), jnp.float32),
            pltpu.SemaphoreType.DMA,
            pltpu.SemaphoreType.DMA,
        ],
        **_SC_PARAMS,
    )
    return fn(zpk, ia, ib)


@jax.jit
def kernel(x, edge_index, edge_label_index, W_l1, b_l1, W_r1, W_l2, b_l2, W_r2):
    n, d_in = x.shape
    d_hid = W_l1.shape[1]
    d_out = W_l2.shape[1]
    e = edge_index.shape[1]
    el = edge_label_index.shape[1]

    epad = EP - e
    packed = (edge_index[1] << 16) | edge_index[0]
    packed = jnp.concatenate(
        [packed, jnp.full((epad,), n << 16, jnp.int32)]).reshape(NW, EPW)
    zeros_hid = jnp.zeros((NPAD, d_hid), jnp.float32)
    zeros_out = jnp.zeros((NPAD, d_out), jnp.float32)

    deg = _sc_degree(packed)
    deg2d = deg.reshape(NC * NPAD, 1)

    Wl1p = W_l1[:, _expand_perm(d_hid)]
    Wl2p = W_l2[:, _expand_perm(d_out)]

    y1, r1 = _tc1(x, Wl1p, W_r1, b_l1)
    agg1 = _sc_aggregate(_pack_bf16(y1), packed, zeros_hid)
    r1p = jnp.concatenate([r1, jnp.zeros((NPAD - n, d_hid), jnp.float32)])
    y2, r2 = _tc2(agg1, deg2d, r1p, Wl2p, W_r2, b_l2)
    agg2 = _sc_aggregate(_pack_bf16(y2), packed, zeros_out)
    z = _tc3(agg2, deg2d, r2)

    pad = ELP - el
    ia = jnp.concatenate([edge_label_index[0], jnp.zeros((pad,), jnp.int32)])
    ib = jnp.concatenate([edge_label_index[1], jnp.zeros((pad,), jnp.int32)])
    out = _sc_decode(_pack_bf16(z), ia, ib)
    return out[:el]


# speedup vs baseline: 1.0455x; 1.0455x over previous
import jax
import jax.numpy as jnp
import numpy as np
from jax import lax
from jax.experimental import pallas as pl
from jax.experimental.pallas import tpu as pltpu
from jax.experimental.pallas import tpu_sc as plsc

NC = 2
NS = 16
NW = NC * NS
LANES = 16

NPAD = 10240
STRIPE = NPAD // NS
ECH = 64
ENCH = 160
EPW = ENCH * ECH
EP = NW * EPW
NBUF = 4
DCH = 64
DNCH = 50
DPW = DNCH * DCH
ELP = NW * DPW

_SC_PARAMS = dict(
    compiler_params=pltpu.CompilerParams(
        use_tc_tiling_on_sc=False, needs_layout_passes=False),
)


def _mesh():
    return plsc.VectorSubcoreMesh(core_axis_name="c", subcore_axis_name="s")


def _expand_perm(d):
    p = np.concatenate([np.arange(0, 32, 2), np.arange(1, 32, 2)])
    p_full = np.concatenate([b * 32 + p for b in range(d // 32)])
    return np.argsort(p_full)


def _pack_bf16(y):
    n, d = y.shape
    yb = y.astype(jnp.bfloat16).reshape(n, d // 2, 2)
    return jax.lax.bitcast_convert_type(yb, jnp.int32)


def _tc1_body(x_ref, wl_ref, wr_ref, b_ref, y_ref, r_ref):
    xv = x_ref[...]
    y_ref[...] = jnp.dot(xv, wl_ref[...], preferred_element_type=jnp.float32)
    r_ref[...] = (
        jnp.dot(xv, wr_ref[...], preferred_element_type=jnp.float32) + b_ref[...]
    )


def _tc1(x, W_l, W_r, b, br=2000):
    n, d = x.shape
    dh = W_l.shape[1]
    grid = n // br
    return pl.pallas_call(
        _tc1_body,
        grid=(grid,),
        in_specs=[
            pl.BlockSpec((br, d), lambda i: (i, 0)),
            pl.BlockSpec((d, dh), lambda i: (0, 0)),
            pl.BlockSpec((d, dh), lambda i: (0, 0)),
            pl.BlockSpec((1, dh), lambda i: (0, 0)),
        ],
        out_specs=[
            pl.BlockSpec((br, dh), lambda i: (i, 0)),
            pl.BlockSpec((br, dh), lambda i: (i, 0)),
        ],
        out_shape=[
            jax.ShapeDtypeStruct((n, dh), jnp.float32),
            jax.ShapeDtypeStruct((n, dh), jnp.float32),
        ],
    )(x, W_l, W_r, b.reshape(1, dh))


def _tc2_body(aggA, aggB, dgA, dgB, r1_ref, wl_ref, wr_ref, b_ref, y2_ref, r2_ref):
    deg = dgA[...] + dgB[...]
    inv = 1.0 / jnp.maximum(deg, 1.0)
    h = jnp.maximum((aggA[...] + aggB[...]) * inv + r1_ref[...], 0.0)
    y2_ref[...] = jnp.dot(h, wl_ref[...], preferred_element_type=jnp.float32)
    r2_ref[...] = (
        jnp.dot(h, wr_ref[...], preferred_element_type=jnp.float32) + b_ref[...]
    )


def _tc2(agg, deg2d, r1, W_l2, W_r2, b_l2, br=2048):
    dh = r1.shape[1]
    do = W_l2.shape[1]
    grid = NPAD // br
    off = NPAD // br
    return pl.pallas_call(
        _tc2_body,
        grid=(grid,),
        in_specs=[
            pl.BlockSpec((br, dh), lambda i: (i, 0)),
            pl.BlockSpec((br, dh), lambda i: (i + off, 0)),
            pl.BlockSpec((br, 1), lambda i: (i, 0)),
            pl.BlockSpec((br, 1), lambda i: (i + off, 0)),
            pl.BlockSpec((br, dh), lambda i: (i, 0)),
            pl.BlockSpec((dh, do), lambda i: (0, 0)),
            pl.BlockSpec((dh, do), lambda i: (0, 0)),
            pl.BlockSpec((1, do), lambda i: (0, 0)),
        ],
        out_specs=[
            pl.BlockSpec((br, do), lambda i: (i, 0)),
            pl.BlockSpec((br, do), lambda i: (i, 0)),
        ],
        out_shape=[
            jax.ShapeDtypeStruct((NPAD, do), jnp.float32),
            jax.ShapeDtypeStruct((NPAD, do), jnp.float32),
        ],
    )(agg, agg, deg2d, deg2d, r1, W_l2, W_r2, b_l2.reshape(1, do))


def _tc3_body(aggA, aggB, dgA, dgB, r2_ref, z_ref):
    deg = dgA[...] + dgB[...]
    inv = 1.0 / jnp.maximum(deg, 1.0)
    z_ref[...] = (aggA[...] + aggB[...]) * inv + r2_ref[...]


def _tc3(agg, deg2d, r2, br=2048):
    do = r2.shape[1]
    grid = NPAD // br
    off = NPAD // br
    return pl.pallas_call(
        _tc3_body,
        grid=(grid,),
        in_specs=[
            pl.BlockSpec((br, do), lambda i: (i, 0)),
            pl.BlockSpec((br, do), lambda i: (i + off, 0)),
            pl.BlockSpec((br, 1), lambda i: (i, 0)),
            pl.BlockSpec((br, 1), lambda i: (i + off, 0)),
            pl.BlockSpec((br, do), lambda i: (i, 0)),
        ],
        out_specs=pl.BlockSpec((br, do), lambda i: (i, 0)),
        out_shape=jax.ShapeDtypeStruct((NPAD, do), jnp.float32),
    )(agg, agg, deg2d, deg2d, r2)


def _sc_degree(packed):
    def body(packed_hbm, deg_out, hist_sh, packed_v, hist_v, part_v, deg_v):
        c = lax.axis_index("c")
        s = lax.axis_index("s")
        wid = c * NS + s
        row0 = s * STRIPE
        pltpu.sync_copy(packed_hbm.at[wid], packed_v)

        zeros16 = jnp.zeros((LANES,), jnp.float32)

        def zero_hist(g, carry):
            hist_v[pl.ds(g * LANES, LANES)] = zeros16
            return carry

        lax.fori_loop(0, NPAD // LANES, zero_hist, 0)

        ones16 = jnp.full((LANES,), 1.0, jnp.float32)

        def count(g, carry):
            p = packed_v[pl.ds(g * LANES, LANES)]
            d_idx = lax.shift_right_logical(p, 16)
            plsc.addupdate_scatter(hist_v, [d_idx], ones16)
            return carry

        lax.fori_loop(0, EPW // LANES, count, 0)

        pltpu.sync_copy(hist_v, hist_sh.at[s])
        plsc.subcore_barrier()
        for t in range(NS):
            pltpu.sync_copy(hist_sh.at[t, pl.ds(row0, STRIPE)], part_v.at[t])

        def merge(g, carry):
            acc = part_v[0, pl.ds(g * LANES, LANES)]
            for t in range(1, NS):
                acc = acc + part_v[t, pl.ds(g * LANES, LANES)]
            deg_v[pl.ds(g * LANES, LANES)] = acc
            return carry

        lax.fori_loop(0, STRIPE // LANES, merge, 0)
        pltpu.sync_copy(deg_v, deg_out.at[pl.ds(c * NPAD + row0, STRIPE)])

    fn = pl.kernel(
        body,
        out_type=jax.ShapeDtypeStruct((NC * NPAD,), jnp.float32),
        mesh=_mesh(),
        scratch_types=[
            pltpu.VMEM_SHARED((NS, NPAD), jnp.float32),
            pltpu.VMEM((EPW,), jnp.int32),
            pltpu.VMEM((NPAD,), jnp.float32),
            pltpu.VMEM((NS, STRIPE), jnp.float32),
            pltpu.VMEM((STRIPE,), jnp.float32),
        ],
        **_SC_PARAMS,
    )
    return fn(packed)


def _sc_aggregate(ypk, packed, zeros_d):
    n, dw = ypk.shape
    d = 2 * dw

    def body(y_hbm, packed_hbm, z_hbm, agg_out, agg_sh, packed_v,
             sidx, didx, rb0, rb1, rb2, rb3, f20, f21,
             gs0, gs1, gs2, gs3, ss0, ss1, ss2, ss3):
        rb = [rb0, rb1, rb2, rb3]
        f2 = [f20, f21]
        gs = [gs0, gs1, gs2, gs3]
        ss = [ss0, ss1, ss2, ss3]
        c = lax.axis_index("c")
        s = lax.axis_index("s")
        wid = c * NS + s
        row0 = s * STRIPE
        pltpu.sync_copy(packed_hbm.at[wid], packed_v)

        def unpack(jc, t):
            for g in range(ECH // LANES):
                p = packed_v[pl.ds(jc * ECH + g * LANES, LANES)]
                sidx[t, pl.ds(g * LANES, LANES)] = p & 0xFFFF
                didx[t, pl.ds(g * LANES, LANES)] = lax.shift_right_logical(p, 16)

        def gather(jc_unused, t):
            pltpu.async_copy(y_hbm.at[sidx.at[t]], rb[t], gs[t])

        def wait_gather(t):
            pltpu.make_async_copy(y_hbm.at[sidx.at[t]], rb[t], gs[t]).wait()

        def convert(t):
            fb = f2[t % 2]
            hi = jnp.int32(-65536)

            def crow(r, carry):
                for q in range(dw // LANES):
                    w = rb[t][r, pl.ds(q * LANES, LANES)]
                    fb[r, pl.ds(q * 2 * LANES, LANES)] = plsc.bitcast(
                        jnp.left_shift(w, 16), jnp.float32)
                    fb[r, pl.ds(q * 2 * LANES + LANES, LANES)] = plsc.bitcast(
                        w & hi, jnp.float32)
                return carry

            lax.fori_loop(0, ECH, crow, 0)

        def scatter(t):
            pltpu.async_copy(f2[t % 2], agg_sh.at[didx.at[t]], ss[t],
                             add=True)

        def wait_scatter(t):
            pltpu.make_async_copy(f2[t % 2], agg_sh.at[didx.at[t]],
                                  ss[t]).wait()

        unpack(0, 0)
        unpack(1, 1)
        gather(0, 0)
        gather(1, 1)
        pltpu.sync_copy(z_hbm.at[pl.ds(row0, STRIPE)],
                        agg_sh.at[pl.ds(row0, STRIPE)])
        plsc.subcore_barrier()

        for b in range(NBUF):
            b2 = (b + 2) % NBUF
            wait_gather(b)
            if b >= 2:
                wait_scatter(b2)
            convert(b)
            scatter(b)
            unpack(b + 2, b2)
            gather(b + 2, b2)

        def visit(k, carry):
            for b in range(NBUF):
                j = NBUF * k + b
                b2 = (b + 2) % NBUF
                wait_gather(b)
                wait_scatter(b2)
                convert(b)
                scatter(b)
                jn = jnp.minimum(j + 2, ENCH - 1)
                unpack(jn, b2)
                gather(jn, b2)
            return carry

        lax.fori_loop(1, ENCH // NBUF, visit, 0)

        wait_gather(0)
        wait_gather(1)
        wait_scatter(2)
        wait_scatter(3)
        plsc.subcore_barrier()
        pltpu.sync_copy(agg_sh.at[pl.ds(row0, STRIPE)],
                        agg_out.at[pl.ds(c * NPAD + row0, STRIPE)])

    fn = pl.kernel(
        body,
        out_type=jax.ShapeDtypeStruct((NC * NPAD, d), jnp.float32),
        mesh=_mesh(),
        scratch_types=[
            pltpu.VMEM_SHARED((NPAD, d), jnp.float32),
            pltpu.VMEM((EPW,), jnp.int32),
            pltpu.VMEM((NBUF, ECH), jnp.int32),
            pltpu.VMEM((NBUF, ECH), jnp.int32),
            pltpu.VMEM((ECH, dw), jnp.int32),
            pltpu.VMEM((ECH, dw), jnp.int32),
            pltpu.VMEM((ECH, dw), jnp.int32),
            pltpu.VMEM((ECH, dw), jnp.int32),
            pltpu.VMEM((ECH, d), jnp.float32),
            pltpu.VMEM((ECH, d), jnp.float32),
            pltpu.SemaphoreType.DMA,
            pltpu.SemaphoreType.DMA,
            pltpu.SemaphoreType.DMA,
            pltpu.SemaphoreType.DMA,
            pltpu.SemaphoreType.DMA,
            pltpu.SemaphoreType.DMA,
            pltpu.SemaphoreType.DMA,
            pltpu.SemaphoreType.DMA,
        ],
        **_SC_PARAMS,
    )
    return fn(ypk, packed, zeros_d)


def _sc_decode(zpk, ia, ib):
    n, dw = zpk.shape

    def body(z_hbm, ia_hbm, ib_hbm, out_hbm, ia_v, ib_v,
             za0, zb0, za1, zb1, out_v, gs0, gs1):
        c = lax.axis_index("c")
        s = lax.axis_index("s")
        wid = c * NS + s
        pltpu.sync_copy(ia_hbm.at[pl.ds(wid * DPW, DPW)], ia_v)
        pltpu.sync_copy(ib_hbm.at[pl.ds(wid * DPW, DPW)], ib_v)
        pltpu.async_copy(z_hbm.at[ia_v.at[pl.ds(0, DCH)]], za0, gs0)
        pltpu.async_copy(z_hbm.at[ib_v.at[pl.ds(0, DCH)]], zb0, gs0)

        def compute(j, za_v, zb_v):
            base = j * DCH
            hi = jnp.int32(-65536)
            for g in range(DCH // LANES):
                rws = g * LANES + lax.iota(jnp.int32, LANES)
                acc = jnp.zeros((LANES,), jnp.float32)
                for q in range(dw):
                    cols = jnp.full((LANES,), q, jnp.int32)
                    wa = plsc.load_gather(za_v, [rws, cols])
                    wb = plsc.load_gather(zb_v, [rws, cols])
                    ae = plsc.bitcast(jnp.left_shift(wa, 16), jnp.float32)
                    be = plsc.bitcast(jnp.left_shift(wb, 16), jnp.float32)
                    ao = plsc.bitcast(wa & hi, jnp.float32)
                    bo = plsc.bitcast(wb & hi, jnp.float32)
                    acc = acc + ae * be + ao * bo
                out_v[pl.ds(base + g * LANES, LANES)] = acc

        def pair(k, carry):
            j0 = 2 * k
            pltpu.make_async_copy(z_hbm.at[ia_v.at[pl.ds(0, DCH)]], za0,
                                  gs0).wait()
            pltpu.make_async_copy(z_hbm.at[ib_v.at[pl.ds(0, DCH)]], zb0,
                                  gs0).wait()
            o1 = (j0 + 1) * DCH
            pltpu.async_copy(z_hbm.at[ia_v.at[pl.ds(o1, DCH)]], za1, gs1)
            pltpu.async_copy(z_hbm.at[ib_v.at[pl.ds(o1, DCH)]], zb1, gs1)
            compute(j0, za0, zb0)
            pltpu.make_async_copy(z_hbm.at[ia_v.at[pl.ds(o1, DCH)]], za1,
                                  gs1).wait()
            pltpu.make_async_copy(z_hbm.at[ib_v.at[pl.ds(o1, DCH)]], zb1,
                                  gs1).wait()
            on = jnp.minimum(j0 + 2, DNCH - 1) * DCH
            pltpu.async_copy(z_hbm.at[ia_v.at[pl.ds(on, DCH)]], za0, gs0)
            pltpu.async_copy(z_hbm.at[ib_v.at[pl.ds(on, DCH)]], zb0, gs0)
            compute(j0 + 1, za1, zb1)
            return carry

        lax.fori_loop(0, DNCH // 2, pair, 0)
        pltpu.make_async_copy(z_hbm.at[ia_v.at[pl.ds(0, DCH)]], za0,
                              gs0).wait()
        pltpu.make_async_copy(z_hbm.at[ib_v.at[pl.ds(0, DCH)]], zb0,
                              gs0).wait()
        pltpu.sync_copy(out_v, out_hbm.at[pl.ds(wid * DPW, DPW)])

    fn = pl.kernel(
        body,
        out_type=jax.ShapeDtypeStruct((ELP,), jnp.float32),
        mesh=_mesh(),
        scratch_types=[
            pltpu.VMEM((DPW,), jnp.int32),
            pltpu.VMEM((DPW,), jnp.int32),
            pltpu.VMEM((DCH, dw), jnp.int32),
            pltpu.VMEM((DCH, dw), jnp.int32),
            pltpu.VMEM((DCH, dw), jnp.int32),
            pltpu.VMEM((DCH, dw), jnp.int32),
            pltpu.VMEM((DPW,), jnp.float32),
            pltpu.SemaphoreType.DMA,
            pltpu.SemaphoreType.DMA,
        ],
        **_SC_PARAMS,
    )
    return fn(zpk, ia, ib)


@jax.jit
def kernel(x, edge_index, edge_label_index, W_l1, b_l1, W_r1, W_l2, b_l2, W_r2):
    n, d_in = x.shape
    d_hid = W_l1.shape[1]
    d_out = W_l2.shape[1]
    e = edge_index.shape[1]
    el = edge_label_index.shape[1]

    epad = EP - e
    packed = (edge_index[1] << 16) | edge_index[0]
    packed = jnp.concatenate(
        [packed, jnp.full((epad,), n << 16, jnp.int32)]).reshape(NW, EPW)
    zeros_hid = jnp.zeros((NPAD, d_hid), jnp.float32)
    zeros_out = jnp.zeros((NPAD, d_out), jnp.float32)

    deg = _sc_degree(packed)
    deg2d = deg.reshape(NC * NPAD, 1)

    Wl1p = W_l1[:, _expand_perm(d_hid)]
    Wl2p = W_l2[:, _expand_perm(d_out)]

    y1, r1 = _tc1(x, Wl1p, W_r1, b_l1)
    agg1 = _sc_aggregate(_pack_bf16(y1), packed, zeros_hid)
    r1p = jnp.concatenate([r1, jnp.zeros((NPAD - n, d_hid), jnp.float32)])
    y2, r2 = _tc2(agg1, deg2d, r1p, Wl2p, W_r2, b_l2)
    agg2 = _sc_aggregate(_pack_bf16(y2), packed, zeros_out)
    z = _tc3(agg2, deg2d, r2)

    pad = ELP - el
    ia = jnp.concatenate([edge_label_index[0], jnp.zeros((pad,), jnp.int32)])
    ib = jnp.concatenate([edge_label_index[1], jnp.zeros((pad,), jnp.int32)])
    out = _sc_decode(_pack_bf16(z), ia, ib)
    return out[:el]
